# baseline (speedup 1.0000x reference)
import functools

import jax
import jax.numpy as jnp
from jax import lax
from jax.experimental import pallas as pl
from jax.experimental.pallas import tpu as pltpu
from jax.experimental.pallas import tpu_sc as plsc

_L = 16
_NC = 2
_NS = 16
_NW = _NC * _NS


def _body(n_rows, n_cols, prob_t_hbm, tgt_hbm, rwd_hbm, out_hbm,
          tgt_v, rwd_v, vals_v, acc_v, sem):
    wid = lax.axis_index("s") * _NC + lax.axis_index("c")
    chunk = n_rows // _NW
    base = wid * chunk
    lane = lax.iota(jnp.int32, _L)
    n_colblk = n_rows // 128

    pltpu.sync_copy(tgt_hbm.at[pl.ds(base, chunk)], tgt_v.at[pl.ds(0, chunk)])
    pltpu.sync_copy(rwd_hbm.at[pl.ds(base, chunk)], rwd_v)

    def fire(j, carry):
        t = tgt_v[pl.ds(j, _L)][0]
        pltpu.async_copy(
            prob_t_hbm.at[pl.ds(t, 1),
                          pl.ds(base + (j // 128) * 128, 128)],
            vals_v.at[pl.ds(j, 1), :], sem)
        return carry
    lax.fori_loop(0, chunk, fire, 0, unroll=8)

    pltpu.make_async_copy(
        prob_t_hbm.at[pl.ds(0, chunk), pl.ds(0, 128)], vals_v, sem).wait()

    acc = jnp.zeros((_L,), jnp.float32)
    for g in range(chunk // _L):
        col = (g % 8) * _L + lane
        v = plsc.load_gather(vals_v, [g * _L + lane, col])
        acc = acc + v * rwd_v[pl.ds(g * _L, _L)]

    acc_v[...] = acc
    pltpu.sync_copy(acc_v, out_hbm.at[pl.ds(wid * _L, _L)])


def kernel(prob, target, reward):
    n_rows, n_cols = prob.shape
    prob_t = prob.T
    tgt = target.astype(jnp.int32)
    chunk = n_rows // _NW

    mesh = plsc.VectorSubcoreMesh(core_axis_name="c", subcore_axis_name="s")
    kern = pl.kernel(
        functools.partial(_body, n_rows, n_cols),
        out_type=jax.ShapeDtypeStruct((_NW * _L,), jnp.float32),
        mesh=mesh,
        compiler_params=pltpu.CompilerParams(needs_layout_passes=False),
        scratch_types=[
            pltpu.VMEM((chunk + _L,), jnp.int32),
            pltpu.VMEM((chunk,), jnp.float32),
            pltpu.VMEM((chunk, 128), jnp.float32),
            pltpu.VMEM((_L,), jnp.float32),
            pltpu.SemaphoreType.DMA,
        ],
    )
    partials = kern(prob_t, tgt, reward)
    return -jnp.sum(partials) / n_rows


# speedup vs baseline: 1.0075x; 1.0075x over previous
import functools

import jax
import jax.numpy as jnp
from jax import lax
from jax.experimental import pallas as pl
from jax.experimental.pallas import tpu as pltpu
from jax.experimental.pallas import tpu_sc as plsc

_L = 16
_NC = 2
_NS = 16
_NW = _NC * _NS


def _body(n_rows, n_cols, prob_t_hbm, tgt_hbm, rwd_hbm, out_hbm,
          tgt_v, rwd_v, vals_v, acc_v, sem):
    wid = lax.axis_index("s") * _NC + lax.axis_index("c")
    chunk = n_rows // _NW
    base = wid * chunk
    lane = lax.iota(jnp.int32, _L)

    pltpu.sync_copy(tgt_hbm.at[pl.ds(base, chunk)], tgt_v.at[pl.ds(0, chunk)])
    pltpu.sync_copy(rwd_hbm.at[pl.ds(base, chunk)], rwd_v)

    def fire(j, carry):
        t = tgt_v[pl.ds(j, _L)][0]
        pltpu.async_copy(
            prob_t_hbm.at[pl.ds(t, 1),
                          pl.ds(base + (j // 128) * 128, 128)],
            vals_v.at[pl.ds(j, 1), :], sem)
        return carry
    lax.fori_loop(0, chunk, fire, 0, unroll=8)

    pltpu.make_async_copy(
        prob_t_hbm.at[pl.ds(0, chunk), pl.ds(0, 128)], vals_v, sem).wait()

    acc = jnp.zeros((_L,), jnp.float32)
    for g in range(chunk // _L):
        col = (g % 8) * _L + lane
        v = plsc.load_gather(vals_v, [g * _L + lane, col])
        acc = acc + v * rwd_v[pl.ds(g * _L, _L)]

    acc_v[...] = acc
    pltpu.sync_copy(acc_v, out_hbm.at[pl.ds(wid * _L, _L)])


def kernel(prob, target, reward):
    n_rows, n_cols = prob.shape
    prob_t = prob.T
    tgt = target.astype(jnp.int32)
    chunk = n_rows // _NW

    mesh = plsc.VectorSubcoreMesh(core_axis_name="c", subcore_axis_name="s")
    kern = pl.kernel(
        functools.partial(_body, n_rows, n_cols),
        out_type=jax.ShapeDtypeStruct((_NW * _L,), jnp.float32),
        mesh=mesh,
        compiler_params=pltpu.CompilerParams(needs_layout_passes=False),
        scratch_types=[
            pltpu.VMEM((chunk + _L,), jnp.int32),
            pltpu.VMEM((chunk,), jnp.float32),
            pltpu.VMEM((chunk, 128), jnp.float32),
            pltpu.VMEM((_L,), jnp.float32),
            pltpu.SemaphoreType.DMA,
        ],
    )
    partials = kern(prob_t, tgt, reward)
    return -jnp.sum(partials) / n_rows
